# baseline (speedup 1.0000x reference)
import jax
import jax.numpy as jnp
from jax.experimental import pallas as pl
from jax.experimental.pallas import tpu as pltpu

_T = 4


def _stitch_kernel(meta_ref, x0_ref, x1_ref, x2_ref, x3_ref,
                   sW_ref, sb_ref, pW_ref, pb_ref, o_ref,
                   sWb, pWb, xg):
    s = pl.program_id(0)
    n_tiles = pl.num_programs(0)
    F = x0_ref.shape[1]

    @pl.when(s == 0)
    def _cast_weights():
        sWb[...] = sW_ref[...].astype(jnp.bfloat16)
        pWb[...] = pW_ref[...].astype(jnp.bfloat16)

    cnt = meta_ref[s]
    e = meta_ref[n_tiles + s]

    slot = s % 2
    @pl.when(cnt > 0)
    def _tile():
        xrefs = (x0_ref, x1_ref, x2_ref, x3_ref)
        for k in range(_T):
            xg[slot, k * F:(k + 1) * F, :] = xrefs[k][0].astype(jnp.bfloat16)
        h = jnp.dot(xg[slot], sWb[e], preferred_element_type=jnp.float32)
        h = h + sb_ref[e]
        h = h / (1.0 + jnp.abs(h))
        og = jnp.dot(h.astype(jnp.bfloat16), pWb[e],
                     preferred_element_type=jnp.float32)
        og = og + pb_ref[e]
        for k in range(_T):
            t = meta_ref[2 * n_tiles + _T * s + k]

            @pl.when(k < cnt)
            def _store():
                o_ref[t] = og[k * F:(k + 1) * F, :]


def kernel(x, eid, stitch_W, stitch_b, proj_W, proj_b):
    B, F, N = x.shape
    E, _, M = stitch_W.shape
    P = proj_W.shape[-1]
    S = B // _T + E - 1

    eid32 = eid.astype(jnp.int32)
    order = jnp.argsort(eid32).astype(jnp.int32)
    counts = jnp.bincount(eid32, length=E).astype(jnp.int32)
    offs = jnp.concatenate([jnp.zeros((1,), jnp.int32),
                            jnp.cumsum(counts).astype(jnp.int32)])
    ntile = -(-counts // _T)
    tstart = jnp.concatenate([jnp.zeros((1,), jnp.int32),
                              jnp.cumsum(ntile).astype(jnp.int32)])
    s_idx = jnp.arange(S, dtype=jnp.int32)
    e_s = jnp.clip(jnp.searchsorted(tstart, s_idx, side='right') - 1, 0, E - 1)
    e_s = e_s.astype(jnp.int32)
    k_s = s_idx - tstart[e_s]
    live = s_idx < tstart[E]
    cnt_s = jnp.where(live, jnp.clip(counts[e_s] - k_s * _T, 0, _T), 0)
    base_s = offs[e_s] + k_s * _T
    slot = base_s[:, None] + jnp.arange(_T, dtype=jnp.int32)[None, :]
    trial = order[jnp.clip(slot, 0, B - 1)]
    meta = jnp.concatenate([cnt_s.astype(jnp.int32), e_s,
                            trial.reshape(-1).astype(jnp.int32)])

    sb3 = stitch_b.reshape(E, 1, M)
    pb3 = proj_b.reshape(E, 1, P)

    x_specs = [
        pl.BlockSpec((1, F, N),
                     (lambda k: (lambda i, m: (m[2 * S + _T * i + k], 0, 0)))(k))
        for k in range(_T)
    ]
    grid_spec = pltpu.PrefetchScalarGridSpec(
        num_scalar_prefetch=1,
        grid=(S,),
        in_specs=x_specs + [
            pl.BlockSpec((E, N, M), lambda i, m: (0, 0, 0)),
            pl.BlockSpec((E, 1, M), lambda i, m: (0, 0, 0)),
            pl.BlockSpec((E, M, P), lambda i, m: (0, 0, 0)),
            pl.BlockSpec((E, 1, P), lambda i, m: (0, 0, 0)),
        ],
        out_specs=pl.BlockSpec((B, F, P), lambda i, m: (0, 0, 0)),
        scratch_shapes=[
            pltpu.VMEM((E, N, M), jnp.bfloat16),
            pltpu.VMEM((E, M, P), jnp.bfloat16),
            pltpu.VMEM((2, _T * F, N), jnp.bfloat16),
        ],
    )
    return pl.pallas_call(
        _stitch_kernel,
        grid_spec=grid_spec,
        out_shape=jax.ShapeDtypeStruct((B, F, P), jnp.float32),
    )(meta, x, x, x, x, stitch_W, sb3, proj_W, pb3)


# speedup vs baseline: 1.5517x; 1.5517x over previous
import jax
import jax.numpy as jnp
from jax.experimental import pallas as pl
from jax.experimental.pallas import tpu as pltpu

_T = 16


def _stitch_kernel(eid_ref, x_ref, sW_ref, sb_ref, pW_ref, pb_ref, o_ref,
                   sWb, pWb):
    i = pl.program_id(0)

    @pl.when(i == 0)
    def _cast_weights():
        sWb[...] = sW_ref[...].astype(jnp.bfloat16)
        pWb[...] = pW_ref[...].astype(jnp.bfloat16)

    for k in range(_T):
        e = eid_ref[i * _T + k]
        xk = x_ref[k].astype(jnp.bfloat16)
        h = jnp.dot(xk, sWb[e], preferred_element_type=jnp.float32)
        h = h + sb_ref[e]
        h = h / (1.0 + jnp.abs(h))
        o = jnp.dot(h.astype(jnp.bfloat16), pWb[e],
                    preferred_element_type=jnp.float32)
        o_ref[k] = o + pb_ref[e]


def kernel(x, eid, stitch_W, stitch_b, proj_W, proj_b):
    B, F, N = x.shape
    E, _, M = stitch_W.shape
    P = proj_W.shape[-1]

    eid32 = eid.astype(jnp.int32)
    sb3 = stitch_b.reshape(E, 1, M)
    pb3 = proj_b.reshape(E, 1, P)

    grid_spec = pltpu.PrefetchScalarGridSpec(
        num_scalar_prefetch=1,
        grid=(B // _T,),
        in_specs=[
            pl.BlockSpec((_T, F, N), lambda i, eid: (i, 0, 0)),
            pl.BlockSpec((E, N, M), lambda i, eid: (0, 0, 0)),
            pl.BlockSpec((E, 1, M), lambda i, eid: (0, 0, 0)),
            pl.BlockSpec((E, M, P), lambda i, eid: (0, 0, 0)),
            pl.BlockSpec((E, 1, P), lambda i, eid: (0, 0, 0)),
        ],
        out_specs=pl.BlockSpec((_T, F, P), lambda i, eid: (i, 0, 0)),
        scratch_shapes=[
            pltpu.VMEM((E, N, M), jnp.bfloat16),
            pltpu.VMEM((E, M, P), jnp.bfloat16),
        ],
    )
    return pl.pallas_call(
        _stitch_kernel,
        grid_spec=grid_spec,
        out_shape=jax.ShapeDtypeStruct((B, F, P), jnp.float32),
    )(eid32, x, stitch_W, sb3, proj_W, pb3)
